# Initial kernel scaffold; baseline (speedup 1.0000x reference)
import jax
import jax.numpy as jnp
from jax.experimental import pallas as pl


def kernel(uc_pairs, conv_data, user_history, arc_in, arc_out, emb_word, emb_user, f2h_W, f2h_b, w3, b3, w4, b4, w5, b5, u_Wih, u_Whh, u_bih, u_bhh, f_Wih, f_Whh, f_bih, f_bhh, b_Wih, b_Whh, b_bih, b_bhh, g_Win, g_Wout, g_b, g_Wg, g_bg, mlp1_W, mlp1_b, mlp2_W, mlp2_b, out_W, out_b):
    raise NotImplementedError("write your pallas kernel here")


# speedup vs baseline: 28.9096x; 28.9096x over previous
import functools

import jax
import jax.numpy as jnp
from jax import lax
from jax.experimental import pallas as pl
from jax.experimental.pallas import tpu as pltpu
from jax.experimental.pallas import tpu_sc as plsc


def _sc_gather(table, idx):
    R = idx.shape[0]
    D = table.shape[1]
    NW = 32
    C = 128
    assert R % (NW * C) == 0, (R, NW * C)
    per_w = R // NW
    chunks = per_w // C
    mesh = plsc.VectorSubcoreMesh(core_axis_name="c", subcore_axis_name="s")

    @functools.partial(
        pl.kernel,
        out_type=jax.ShapeDtypeStruct((R, D), table.dtype),
        mesh=mesh,
        scratch_types=[
            pltpu.VMEM((C,), jnp.int32),
            pltpu.VMEM((C,), jnp.int32),
            pltpu.VMEM((C, D), table.dtype),
            pltpu.VMEM((C, D), table.dtype),
            pltpu.SemaphoreType.DMA,
            pltpu.SemaphoreType.DMA,
        ],
    )
    def k(table_hbm, idx_hbm, out_hbm, idx_a, idx_b, rows_a, rows_b, sem_a, sem_b):
        wid = lax.axis_index("s") * 2 + lax.axis_index("c")
        base = wid * per_w

        def body(i, carry):
            off = base + i * C

            @pl.when(i % 2 == 0)
            def _even():
                pltpu.sync_copy(idx_hbm.at[pl.ds(off, C)], idx_a)
                pltpu.async_copy(table_hbm.at[idx_a], rows_a, sem_a).wait()
                pltpu.sync_copy(rows_a, out_hbm.at[pl.ds(off, C)])

            @pl.when(i % 2 == 1)
            def _odd():
                pltpu.sync_copy(idx_hbm.at[pl.ds(off, C)], idx_b)
                pltpu.async_copy(table_hbm.at[idx_b], rows_b, sem_b).wait()
                pltpu.sync_copy(rows_b, out_hbm.at[pl.ds(off, C)])

            return carry

        lax.fori_loop(0, chunks, body, 0)

    return k(table, idx)


def _cnn_kernel(ge_ref, wcat_ref, b3_ref, b4_ref, b5_ref, out_ref, p_scr, *, M, L, K):
    for l in range(L):
        x = ge_ref[:, l, :]
        p_scr[:, l, :] = jnp.dot(x, wcat_ref[...],
                                 preferred_element_type=jnp.float32)

    outs = []
    col = 0
    for ksz, b_ref in ((3, b3_ref), (4, b4_ref), (5, b5_ref)):
        n = L - ksz + 1
        acc = p_scr[:, 0:n, col:col + K]
        for i in range(1, ksz):
            acc = acc + p_scr[:, i:i + n, col + i * K:col + (i + 1) * K]
        col += ksz * K
        acc = jnp.maximum(acc + b_ref[0], 0.0)
        outs.append(jnp.max(acc, axis=1))
    out_ref[...] = jnp.concatenate(outs, axis=-1)


def _msg_cnn(ge, wcat, b3, b4, b5):
    N, L, E = ge.shape
    K = b3.shape[-1]
    M = 256
    while N % M:
        M //= 2
    grid = (N // M,)
    return pl.pallas_call(
        functools.partial(_cnn_kernel, M=M, L=L, K=K),
        grid=grid,
        in_specs=[
            pl.BlockSpec((M, L, E), lambda i: (i, 0, 0)),
            pl.BlockSpec(wcat.shape, lambda i: (0, 0)),
            pl.BlockSpec(b3.shape, lambda i: (0, 0)),
            pl.BlockSpec(b4.shape, lambda i: (0, 0)),
            pl.BlockSpec(b5.shape, lambda i: (0, 0)),
        ],
        out_specs=pl.BlockSpec((M, 3 * K), lambda i: (i, 0)),
        out_shape=jax.ShapeDtypeStruct((N, 3 * K), jnp.float32),
        scratch_shapes=[pltpu.VMEM((M, L, 12 * K), jnp.float32)],
        compiler_params=pltpu.CompilerParams(
            vmem_limit_bytes=100 * 1024 * 1024),
    )(ge, wcat, b3, b4, b5)


def _gru_step(x_parts, h, m, WhhT, bih, bhh, HH):
    gi = bih[0]
    for xp, wp in x_parts:
        gi = gi + jnp.dot(xp, wp, preferred_element_type=jnp.float32)
    gh = jnp.dot(h, WhhT, preferred_element_type=jnp.float32) + bhh[0]
    ir, iz, inn = gi[:, :HH], gi[:, HH:2 * HH], gi[:, 2 * HH:]
    hr, hz, hn = gh[:, :HH], gh[:, HH:2 * HH], gh[:, 2 * HH:]
    r = jax.nn.sigmoid(ir + hr)
    z = jax.nn.sigmoid(iz + hz)
    n = jnp.tanh(inn + r * hn)
    hnew = (1.0 - z) * n + z * h
    return m * hnew + (1.0 - m) * h


def _seq_kernel(hrep_ref, hmask_ref, uembh_ref, cnnt_ref, uembt_ref,
                tmask_ref, ainT_ref, aoutT_ref,
                f2hW_ref, f2hb_ref,
                uWihT_ref, uWhhT_ref, ubih_ref, ubhh_ref,
                fWa_ref, fWb_ref, fWhhT_ref, fbih_ref, fbhh_ref,
                bWa_ref, bWb_ref, bWhhT_ref, bbih_ref, bbhh_ref,
                gWin_ref, gWout_ref, gb_ref, gWg_ref, gbg_ref,
                m1W_ref, m1b_ref, m2W_ref, m2b_ref, outWT_ref, outb_ref,
                out_ref, H_scr, *, Bb, T, HIST, HID, NL):
    HH = HID // 2

    h = jnp.tanh(jnp.dot(uembh_ref[...], f2hW_ref[...],
                         preferred_element_type=jnp.float32) + f2hb_ref[0])
    for t in range(HIST):
        x = hrep_ref[:, t, :]
        m = hmask_ref[:, t:t + 1]
        h = _gru_step([(x, uWihT_ref[...])], h, m,
                      uWhhT_ref[...], ubih_ref, ubhh_ref, HID)
    user_out = h

    def bi(dirn, Wa_ref, Wb_ref, WhhT_ref, bih_ref, bhh_ref, colo):
        hh = jnp.zeros((Bb, HH), jnp.float32)
        steps = range(T) if dirn > 0 else range(T - 1, -1, -1)
        for t in steps:
            m = tmask_ref[:, t:t + 1]
            xu = uembt_ref[:, t, :]
            xc = cnnt_ref[:, t, :]
            gi = (jnp.dot(xu, Wa_ref[...], preferred_element_type=jnp.float32)
                  + jnp.dot(xc, Wb_ref[...], preferred_element_type=jnp.float32))
            gi = m * gi + bih_ref[0]
            gh = jnp.dot(hh, WhhT_ref[...],
                         preferred_element_type=jnp.float32) + bhh_ref[0]
            ir, iz, inn = gi[:, :HH], gi[:, HH:2 * HH], gi[:, 2 * HH:]
            hr, hz, hn = gh[:, :HH], gh[:, HH:2 * HH], gh[:, 2 * HH:]
            r = jax.nn.sigmoid(ir + hr)
            z = jax.nn.sigmoid(iz + hz)
            n = jnp.tanh(inn + r * hn)
            hnew = (1.0 - z) * n + z * hh
            hh = m * hnew + (1.0 - m) * hh
            H_scr[:, t, colo:colo + HH] = hh

    bi(+1, fWa_ref, fWb_ref, fWhhT_ref, fbih_ref, fbhh_ref, 0)
    bi(-1, bWa_ref, bWb_ref, bWhhT_ref, bbih_ref, bbhh_ref, HH)

    tm3 = tmask_ref[...][:, :, None]
    H = H_scr[...] * tm3

    for lyr in range(NL):
        m1 = jnp.zeros((Bb, T, HID), jnp.float32)
        m2 = jnp.zeros((Bb, T, HID), jnp.float32)
        for s in range(T):
            hs = H[:, s, :][:, None, :]
            m1 = m1 + ainT_ref[:, s, :][:, :, None] * hs
            m2 = m2 + aoutT_ref[:, s, :][:, :, None] * hs
        m1f = m1.reshape(Bb * T, HID)
        m2f = m2.reshape(Bb * T, HID)
        Hf = H.reshape(Bb * T, HID)
        mm = (jnp.dot(m1f, gWin_ref[lyr], preferred_element_type=jnp.float32)
              + jnp.dot(m2f, gWout_ref[lyr], preferred_element_type=jnp.float32)
              + gb_ref[lyr])
        a = jnp.maximum(mm, 0.0)
        g = jax.nn.sigmoid(jnp.dot(Hf, gWg_ref[lyr],
                                   preferred_element_type=jnp.float32)
                           + gbg_ref[lyr])
        H = (g * a + (1.0 - g) * Hf).reshape(Bb, T, HID)

    H = H * tm3

    scores = jnp.sum(H * user_out[:, None, :], axis=2) + (1.0 - tmask_ref[...]) * (-1e9)
    mx = jnp.max(scores, axis=1, keepdims=True)
    e = jnp.exp(scores - mx)
    aw = e / jnp.sum(e, axis=1, keepdims=True)
    fin = jnp.sum(H * aw[:, :, None], axis=1)

    o = jnp.maximum(
        jnp.dot(fin, m1W_ref[...][:HID], preferred_element_type=jnp.float32)
        + jnp.dot(user_out, m1W_ref[...][HID:], preferred_element_type=jnp.float32)
        + m1b_ref[0], 0.0)
    o = jnp.maximum(jnp.dot(o, m2W_ref[...],
                            preferred_element_type=jnp.float32) + m2b_ref[0], 0.0)
    val = jnp.sum(o * outWT_ref[0], axis=1, keepdims=True) + outb_ref[0]
    out_ref[...] = jnp.broadcast_to(val, (Bb, 128))


def _seq_stage(hrep, hmask, uembh, cnnt, uembt, tmask, ainT, aoutT, params):
    B, HIST, _ = hrep.shape
    T = tmask.shape[1]
    HID = params["f2hW"].shape[1]
    NL = params["gWin"].shape[0]
    Bb = 128
    while B % Bb:
        Bb //= 2
    grid = (B // Bb,)

    w_names = ("f2hW", "f2hb", "uWihT", "uWhhT", "ubih", "ubhh",
               "fWa", "fWb", "fWhhT", "fbih", "fbhh",
               "bWa", "bWb", "bWhhT", "bbih", "bbhh",
               "gWin", "gWout", "gb", "gWg", "gbg",
               "m1W", "m1b", "m2W", "m2b", "outWT", "outb")
    ws = [params[n] for n in w_names]

    def full(a):
        nd = a.ndim
        return pl.BlockSpec(a.shape, lambda i, _nd=nd: (0,) * _nd)

    in_specs = [
        pl.BlockSpec((Bb, HIST, hrep.shape[2]), lambda i: (i, 0, 0)),
        pl.BlockSpec((Bb, HIST), lambda i: (i, 0)),
        pl.BlockSpec((Bb, uembh.shape[1]), lambda i: (i, 0)),
        pl.BlockSpec((Bb, T, cnnt.shape[2]), lambda i: (i, 0, 0)),
        pl.BlockSpec((Bb, T, uembt.shape[2]), lambda i: (i, 0, 0)),
        pl.BlockSpec((Bb, T), lambda i: (i, 0)),
        pl.BlockSpec((Bb, T, T), lambda i: (i, 0, 0)),
        pl.BlockSpec((Bb, T, T), lambda i: (i, 0, 0)),
    ] + [full(w) for w in ws]

    out = pl.pallas_call(
        functools.partial(_seq_kernel, Bb=Bb, T=T, HIST=HIST, HID=HID, NL=NL),
        grid=grid,
        in_specs=in_specs,
        out_specs=pl.BlockSpec((Bb, 128), lambda i: (i, 0)),
        out_shape=jax.ShapeDtypeStruct((B, 128), jnp.float32),
        scratch_shapes=[pltpu.VMEM((Bb, T, HID), jnp.float32)],
        compiler_params=pltpu.CompilerParams(
            vmem_limit_bytes=100 * 1024 * 1024),
    )(hrep, hmask, uembh, cnnt, uembt, tmask, ainT, aoutT, *ws)
    return out[:, 0]


def kernel(uc_pairs, conv_data, user_history, arc_in, arc_out, emb_word,
           emb_user, f2h_W, f2h_b, w3, b3, w4, b4, w5, b5,
           u_Wih, u_Whh, u_bih, u_bhh,
           f_Wih, f_Whh, f_bih, f_bhh,
           b_Wih, b_Whh, b_bih, b_bhh,
           g_Win, g_Wout, g_b, g_Wg, g_bg,
           mlp1_W, mlp1_b, mlp2_W, mlp2_b, out_W, out_b):
    B = uc_pairs.shape[0]
    CONV, T, W4L = conv_data.shape
    L = W4L - 4
    HIST = user_history.shape[1]
    E = emb_word.shape[1]
    K = b3.shape[0]
    HID = f2h_W.shape[1]

    userids = uc_pairs[:, 0]
    convids = uc_pairs[:, 1]

    hist = jnp.take(user_history, userids, axis=0)
    convs = jnp.take(conv_data, convids, axis=0)
    t0 = convs[:, 0, 1]
    hmask = (hist[:, :, 1] < t0[:, None]).astype(jnp.float32)
    tmask = (convs[:, :, 0] >= 0).astype(jnp.float32)
    uids = jnp.maximum(convs[:, :, 0], 0)

    cdm = conv_data.reshape(CONV * T, 4 + L)
    hist_msg = hist[:, :, 0] * T + hist[:, :, 2]
    hw = jnp.take(cdm, hist_msg.reshape(-1), axis=0)[:, 4:]
    words_conv = convs[:, :, 4:].reshape(B * T, L)
    all_words = jnp.concatenate([hw.reshape(-1), words_conv.reshape(-1)])

    ge = _sc_gather(emb_word, all_words.astype(jnp.int32))
    ge = ge.reshape(B * (HIST + T), L, E)

    wcat = jnp.concatenate(
        [w3[i] for i in range(3)] + [w4[i] for i in range(4)]
        + [w5[i] for i in range(5)], axis=1)
    reps = _msg_cnn(ge, wcat, b3.reshape(1, K), b4.reshape(1, K),
                    b5.reshape(1, K))
    hrep = reps[:B * HIST].reshape(B, HIST, 3 * K)
    cnnt = reps[B * HIST:].reshape(B, T, 3 * K)

    uembh = jnp.take(emb_user, userids, axis=0)
    uembt = jnp.take(emb_user, uids.reshape(-1), axis=0).reshape(B, T, -1)
    ainT = jnp.swapaxes(jnp.take(arc_in, convids, axis=0), 1, 2)
    aoutT = jnp.swapaxes(jnp.take(arc_out, convids, axis=0), 1, 2)

    F = emb_user.shape[1]
    fWihT = f_Wih.T
    bWihT = b_Wih.T
    params = dict(
        f2hW=f2h_W, f2hb=f2h_b.reshape(1, HID),
        uWihT=u_Wih.T, uWhhT=u_Whh.T,
        ubih=u_bih.reshape(1, -1), ubhh=u_bhh.reshape(1, -1),
        fWa=fWihT[:F], fWb=fWihT[F:], fWhhT=f_Whh.T,
        fbih=f_bih.reshape(1, -1), fbhh=f_bhh.reshape(1, -1),
        bWa=bWihT[:F], bWb=bWihT[F:], bWhhT=b_Whh.T,
        bbih=b_bih.reshape(1, -1), bbhh=b_bhh.reshape(1, -1),
        gWin=g_Win, gWout=g_Wout, gb=g_b.reshape(g_b.shape[0], 1, HID),
        gWg=g_Wg, gbg=g_bg.reshape(g_bg.shape[0], 1, HID),
        m1W=mlp1_W, m1b=mlp1_b.reshape(1, -1),
        m2W=mlp2_W, m2b=mlp2_b.reshape(1, -1),
        outWT=out_W.T, outb=out_b.reshape(1, 1),
    )
    return _seq_stage(hrep, hmask, uembh, cnnt, uembt, tmask, ainT, aoutT,
                      params)
